# baseline (speedup 1.0000x reference)
import jax
import jax.numpy as jnp
from jax.experimental import pallas as pl
from jax.experimental.pallas import tpu as pltpu

_SRC_X = 2048
_SRC_Y = 2048
_ROWS = 4096
_COLS = 4096
_NCHUNK = 4
_CHUNK = _ROWS // _NCHUNK
_PR = 8
_PC = 128


def _body(b_hbm, bt_smem, o_hbm, patch, sem_big, sem_small):
    copies = []
    for i in range(_NCHUNK):
        cp = pltpu.make_async_copy(
            b_hbm.at[:, pl.ds(i * _CHUNK, _CHUNK), :],
            o_hbm.at[:, pl.ds(i * _CHUNK, _CHUNK), :],
            sem_big,
        )
        cp.start()
        copies.append(cp)

    cp_in = pltpu.make_async_copy(
        b_hbm.at[:, pl.ds(_SRC_X, _PR), pl.ds(_SRC_Y, _PC)], patch, sem_small
    )
    cp_in.start()
    cp_in.wait()
    row_ids = jax.lax.broadcasted_iota(jnp.int32, (_PR, _PC), 0)
    col_ids = jax.lax.broadcasted_iota(jnp.int32, (_PR, _PC), 1)
    mask = (row_ids == 0) & (col_ids == 0)
    patch[0] = jnp.where(mask, bt_smem[0, 0], patch[0])

    for cp in copies:
        cp.wait()

    cp_out = pltpu.make_async_copy(
        patch, o_hbm.at[:, pl.ds(_SRC_X, _PR), pl.ds(_SRC_Y, _PC)], sem_small
    )
    cp_out.start()
    cp_out.wait()


def kernel(B, Bt):
    return pl.pallas_call(
        _body,
        in_specs=[
            pl.BlockSpec(memory_space=pl.ANY),
            pl.BlockSpec(memory_space=pltpu.SMEM),
        ],
        out_specs=pl.BlockSpec(memory_space=pl.ANY),
        out_shape=jax.ShapeDtypeStruct((1, _ROWS, _COLS), jnp.float32),
        scratch_shapes=[
            pltpu.VMEM((1, _PR, _PC), jnp.float32),
            pltpu.SemaphoreType.DMA,
            pltpu.SemaphoreType.DMA,
        ],
    )(B, Bt)


# speedup vs baseline: 46.5716x; 46.5716x over previous
import jax
import jax.numpy as jnp
from jax.experimental import pallas as pl
from jax.experimental.pallas import tpu as pltpu

_SRC_X = 2048
_SRC_Y = 2048
_ROWS = 4096
_COLS = 4096
_BLK = 256


def _copy_scatter_kernel(bt_ref, b_ref, o_ref):
    i = pl.program_id(0)
    o_ref[...] = b_ref[...]

    @pl.when(i == _SRC_X // _BLK)
    def _():
        r = _SRC_X % _BLK
        row = b_ref[0, r : r + 1, :]
        col_ids = jax.lax.broadcasted_iota(jnp.int32, (1, _COLS), 1)
        o_ref[0, r : r + 1, :] = jnp.where(col_ids == _SRC_Y, bt_ref[0, 0], row)


def kernel(B, Bt):
    return pl.pallas_call(
        _copy_scatter_kernel,
        grid=(_ROWS // _BLK,),
        in_specs=[
            pl.BlockSpec(memory_space=pltpu.SMEM),
            pl.BlockSpec((1, _BLK, _COLS), lambda i: (0, i, 0)),
        ],
        out_specs=pl.BlockSpec((1, _BLK, _COLS), lambda i: (0, i, 0)),
        out_shape=jax.ShapeDtypeStruct((1, _ROWS, _COLS), jnp.float32),
    )(Bt, B)
